# Initial kernel scaffold; baseline (speedup 1.0000x reference)
import jax
import jax.numpy as jnp
from jax.experimental import pallas as pl


def kernel(pos, vel, edge_index):
    raise NotImplementedError("write your pallas kernel here")


# speedup vs baseline: 21.8683x; 21.8683x over previous
import dataclasses

import jax
import jax.numpy as jnp
from jax import lax
from jax.experimental import pallas as pl
from jax.experimental.pallas import tpu as pltpu
from jax.experimental.pallas import tpu_sc as plsc

N_NODES = 100000
N_EDGES = 6400000

NC = 2
NS = 16
NW = NC * NS

EDGES_PER_W = N_EDGES // NW
BLK = 2000
NBLK = EDGES_PER_W // BLK
SEG = 80
NSEG = BLK // SEG
NSTEP = BLK // 16

NPAD = 100352
ROWS_PER_TILE = NPAD // NS
TBL_PER_TILE = NPAD // NS
FIN_PER_W = NPAD // NW
FIN_STEP = FIN_PER_W // 16

_mesh = plsc.VectorSubcoreMesh(core_axis_name="c", subcore_axis_name="s")

_cparams = pltpu.CompilerParams()
for _f, _v in (("needs_layout_passes", False), ("use_tc_tiling_on_sc", False)):
    if _f in pltpu.CompilerParams.__dataclass_fields__:
        _cparams = dataclasses.replace(_cparams, **{_f: _v})


def _edge_kernel(h_hbm, src_hbm, dst_hbm, zrows_hbm, minit_hbm,
                 acc0_hbm, acc1_hbm,
                 idx_s, idx_d, rows_s, rows_d, msg, acc_sh):
    c = lax.axis_index("c")
    s = lax.axis_index("s")
    wid = c * NS + s

    pltpu.sync_copy(zrows_hbm, acc_sh.at[pl.ds(s * ROWS_PER_TILE, ROWS_PER_TILE)])
    pltpu.sync_copy(minit_hbm, msg)
    plsc.subcore_barrier()

    iota = lax.iota(jnp.int32, 16)

    @pl.loop(0, NBLK)
    def _block(i):
        blk = wid * NBLK + i
        pltpu.sync_copy(src_hbm.at[blk], idx_s)
        pltpu.sync_copy(dst_hbm.at[blk], idx_d)

        @pl.loop(0, NSEG)
        def _gather(j):
            off = pl.multiple_of(j * SEG, SEG)
            pltpu.sync_copy(h_hbm.at[idx_s.at[j]], rows_s.at[pl.ds(off, SEG)])
            pltpu.sync_copy(h_hbm.at[idx_d.at[j]], rows_d.at[pl.ds(off, SEG)])

        @pl.loop(0, NSTEP)
        def _compute(k):
            r = k * 16 + iota
            s0 = plsc.load_gather(rows_s, [r, jnp.zeros((16,), jnp.int32)])
            s1 = plsc.load_gather(rows_s, [r, jnp.full((16,), 1, jnp.int32)])
            s2 = plsc.load_gather(rows_s, [r, jnp.full((16,), 2, jnp.int32)])
            s3 = plsc.load_gather(rows_s, [r, jnp.full((16,), 3, jnp.int32)])
            d0 = plsc.load_gather(rows_d, [r, jnp.zeros((16,), jnp.int32)])
            d1 = plsc.load_gather(rows_d, [r, jnp.full((16,), 1, jnp.int32)])
            d2 = plsc.load_gather(rows_d, [r, jnp.full((16,), 2, jnp.int32)])
            d3 = plsc.load_gather(rows_d, [r, jnp.full((16,), 3, jnp.int32)])
            px = s0 - d0
            py = s1 - d1
            vx = s2 - d2
            vy = s3 - d3
            nsq = px * px + py * py
            inv = jnp.float32(1.0) / nsq
            zero = nsq == jnp.float32(0.0)
            cx = jnp.where(zero, jnp.float32(0.0), px * jnp.float32(-5.0) * inv)
            cy = jnp.where(zero, jnp.float32(0.0), py * jnp.float32(-5.0) * inv)
            mx = px * jnp.float32(0.05) + vx
            my = py * jnp.float32(0.05) + vy
            plsc.store_scatter(msg, [r, jnp.zeros((16,), jnp.int32)], mx)
            plsc.store_scatter(msg, [r, jnp.full((16,), 1, jnp.int32)], my)
            plsc.store_scatter(msg, [r, jnp.full((16,), 2, jnp.int32)], cx)
            plsc.store_scatter(msg, [r, jnp.full((16,), 3, jnp.int32)], cy)

        @pl.loop(0, NSEG)
        def _scatter(j):
            off = pl.multiple_of(j * SEG, SEG)
            pltpu.sync_copy(msg.at[pl.ds(off, SEG)], acc_sh.at[idx_d.at[j]],
                            add=True)

    plsc.subcore_barrier()

    @pl.when(c == 0)
    def _out0():
        pltpu.sync_copy(acc_sh.at[pl.ds(s * ROWS_PER_TILE, ROWS_PER_TILE)],
                        acc0_hbm.at[pl.ds(s * ROWS_PER_TILE, ROWS_PER_TILE)])

    @pl.when(c == 1)
    def _out1():
        pltpu.sync_copy(acc_sh.at[pl.ds(s * ROWS_PER_TILE, ROWS_PER_TILE)],
                        acc1_hbm.at[pl.ds(s * ROWS_PER_TILE, ROWS_PER_TILE)])


def _final_kernel(acc0_hbm, acc1_hbm, out_hbm, a0, a1, ob):
    c = lax.axis_index("c")
    s = lax.axis_index("s")
    wid = c * NS + s
    r0 = wid * FIN_PER_W

    pltpu.sync_copy(acc0_hbm.at[pl.ds(r0, FIN_PER_W)], a0)
    pltpu.sync_copy(acc1_hbm.at[pl.ds(r0, FIN_PER_W)], a1)

    iota = lax.iota(jnp.int32, 16)

    @pl.loop(0, FIN_STEP)
    def _step(k):
        r = k * 16 + iota
        cols = [jnp.full((16,), j, jnp.int32) for j in range(5)]
        mx = plsc.load_gather(a0, [r, cols[0]]) + plsc.load_gather(a1, [r, cols[0]])
        my = plsc.load_gather(a0, [r, cols[1]]) + plsc.load_gather(a1, [r, cols[1]])
        cx = plsc.load_gather(a0, [r, cols[2]]) + plsc.load_gather(a1, [r, cols[2]])
        cy = plsc.load_gather(a0, [r, cols[3]]) + plsc.load_gather(a1, [r, cols[3]])
        cnt = plsc.load_gather(a0, [r, cols[4]]) + plsc.load_gather(a1, [r, cols[4]])
        cnt = jnp.maximum(cnt, jnp.float32(1.0))
        ox = cx + mx / cnt
        oy = cy + my / cnt
        plsc.store_scatter(ob, [r, jnp.zeros((16,), jnp.int32)], ox)
        plsc.store_scatter(ob, [r, jnp.full((16,), 1, jnp.int32)], oy)

    pltpu.sync_copy(ob, out_hbm.at[pl.ds(r0, FIN_PER_W)])


@jax.jit
def _run(h, src2d, dst2d):
    zrows = jnp.zeros((ROWS_PER_TILE, 8), jnp.float32)
    minit = jnp.zeros((BLK, 8), jnp.float32).at[:, 4].set(1.0)

    edge_k = pl.kernel(
        _edge_kernel,
        out_type=[jax.ShapeDtypeStruct((NPAD, 8), jnp.float32),
                  jax.ShapeDtypeStruct((NPAD, 8), jnp.float32)],
        mesh=_mesh,
        scratch_types=[
            pltpu.VMEM((NSEG, SEG), jnp.int32),
            pltpu.VMEM((NSEG, SEG), jnp.int32),
            pltpu.VMEM((BLK, 8), jnp.float32),
            pltpu.VMEM((BLK, 8), jnp.float32),
            pltpu.VMEM((BLK, 8), jnp.float32),
            pltpu.VMEM_SHARED((NPAD, 8), jnp.float32),
        ],
        compiler_params=_cparams,
    )
    acc0, acc1 = edge_k(h, src2d, dst2d, zrows, minit)

    final_k = pl.kernel(
        _final_kernel,
        out_type=jax.ShapeDtypeStruct((NPAD, 2), jnp.float32),
        mesh=_mesh,
        scratch_types=[
            pltpu.VMEM((FIN_PER_W, 8), jnp.float32),
            pltpu.VMEM((FIN_PER_W, 8), jnp.float32),
            pltpu.VMEM((FIN_PER_W, 2), jnp.float32),
        ],
        compiler_params=_cparams,
    )
    return final_k(acc0, acc1)


def kernel(pos, vel, edge_index):
    h = jnp.concatenate([pos, vel], axis=-1)
    h = jnp.pad(h, ((0, NPAD - N_NODES), (0, 4)))
    src2d = edge_index[0].reshape(NW * NBLK, NSEG, SEG)
    dst2d = edge_index[1].reshape(NW * NBLK, NSEG, SEG)
    out = _run(h, src2d, dst2d)
    return out[:N_NODES]


# speedup vs baseline: 52.5959x; 2.4051x over previous
import dataclasses

import jax
import jax.numpy as jnp
from jax import lax
from jax.experimental import pallas as pl
from jax.experimental.pallas import tpu as pltpu
from jax.experimental.pallas import tpu_sc as plsc

N_NODES = 100000
N_EDGES = 6400000

NC = 2
NS = 16
NW = NC * NS

EDGES_PER_W = N_EDGES // NW
BLK = 800
NBLK = EDGES_PER_W // BLK
SEG = 100
NSEG = BLK // SEG
NSTEP = BLK // 16

NPAD = 100352
ROWS_PER_TILE = NPAD // NS
TBL_PER_TILE = NPAD // NS
FIN_PER_W = NPAD // NW
FIN_STEP = FIN_PER_W // 16

_mesh = plsc.VectorSubcoreMesh(core_axis_name="c", subcore_axis_name="s")

_cparams = pltpu.CompilerParams()
for _f, _v in (("needs_layout_passes", False), ("use_tc_tiling_on_sc", False)):
    if _f in pltpu.CompilerParams.__dataclass_fields__:
        _cparams = dataclasses.replace(_cparams, **{_f: _v})


NPAIR = NBLK // 2


def _edge_kernel(h_hbm, src_hbm, dst_hbm, zrows_hbm, minit_hbm,
                 acc0_hbm, acc1_hbm,
                 idx_sA, idx_dA, idx_sB, idx_dB,
                 rows_sA, rows_dA, rows_sB, rows_dB,
                 msgA, msgB, acc_sh,
                 semGA, semGB, semSA, semSB):
    c = lax.axis_index("c")
    s = lax.axis_index("s")
    wid = c * NS + s

    pltpu.sync_copy(zrows_hbm, acc_sh.at[pl.ds(s * ROWS_PER_TILE, ROWS_PER_TILE)])
    pltpu.sync_copy(minit_hbm, msgA)
    pltpu.sync_copy(minit_hbm, msgB)
    plsc.subcore_barrier()

    iota = lax.iota(jnp.int32, 16)

    def load_idx(blk, idx_s, idx_d):
        pltpu.sync_copy(src_hbm.at[blk], idx_s)
        pltpu.sync_copy(dst_hbm.at[blk], idx_d)

    def fire_gathers(idx_s, idx_d, rows_s, rows_d, sem):
        @pl.loop(0, NSEG)
        def _g(j):
            off = pl.multiple_of(j * SEG, SEG)
            pltpu.async_copy(h_hbm.at[idx_s.at[j]], rows_s.at[pl.ds(off, SEG)], sem)
            pltpu.async_copy(h_hbm.at[idx_d.at[j]], rows_d.at[pl.ds(off, SEG)], sem)

    def drain(buf, sem):
        pltpu.make_async_copy(h_hbm.at[pl.ds(0, BLK)], buf, sem).wait()

    def compute(rows_s, rows_d, msg):
        @pl.loop(0, NSTEP)
        def _compute(k):
            r = k * 16 + iota
            s0 = plsc.load_gather(rows_s, [r, jnp.zeros((16,), jnp.int32)])
            s1 = plsc.load_gather(rows_s, [r, jnp.full((16,), 1, jnp.int32)])
            s2 = plsc.load_gather(rows_s, [r, jnp.full((16,), 2, jnp.int32)])
            s3 = plsc.load_gather(rows_s, [r, jnp.full((16,), 3, jnp.int32)])
            d0 = plsc.load_gather(rows_d, [r, jnp.zeros((16,), jnp.int32)])
            d1 = plsc.load_gather(rows_d, [r, jnp.full((16,), 1, jnp.int32)])
            d2 = plsc.load_gather(rows_d, [r, jnp.full((16,), 2, jnp.int32)])
            d3 = plsc.load_gather(rows_d, [r, jnp.full((16,), 3, jnp.int32)])
            px = s0 - d0
            py = s1 - d1
            vx = s2 - d2
            vy = s3 - d3
            nsq = px * px + py * py
            inv = jnp.float32(1.0) / nsq
            zero = nsq == jnp.float32(0.0)
            cx = jnp.where(zero, jnp.float32(0.0), px * jnp.float32(-5.0) * inv)
            cy = jnp.where(zero, jnp.float32(0.0), py * jnp.float32(-5.0) * inv)
            mx = px * jnp.float32(0.05) + vx
            my = py * jnp.float32(0.05) + vy
            plsc.store_scatter(msg, [r, jnp.zeros((16,), jnp.int32)], mx)
            plsc.store_scatter(msg, [r, jnp.full((16,), 1, jnp.int32)], my)
            plsc.store_scatter(msg, [r, jnp.full((16,), 2, jnp.int32)], cx)
            plsc.store_scatter(msg, [r, jnp.full((16,), 3, jnp.int32)], cy)

    def fire_scatter(msg, idx_d, sem):
        @pl.loop(0, NSEG)
        def _s(j):
            off = pl.multiple_of(j * SEG, SEG)
            pltpu.async_copy(msg.at[pl.ds(off, SEG)], acc_sh.at[idx_d.at[j]],
                             sem, add=True)

    base = wid * NBLK
    load_idx(base + 0, idx_sA, idx_dA)
    fire_gathers(idx_sA, idx_dA, rows_sA, rows_dA, semGA)
    load_idx(base + 1, idx_sB, idx_dB)
    fire_gathers(idx_sB, idx_dB, rows_sB, rows_dB, semGB)

    @pl.loop(0, NPAIR)
    def _pair(i):
        b0 = base + 2 * i
        drain(rows_sA, semGA)
        drain(rows_dA, semGA)
        compute(rows_sA, rows_dA, msgA)
        fire_scatter(msgA, idx_dA, semSA)
        drain(rows_sB, semGB)
        drain(rows_dB, semGB)
        compute(rows_sB, rows_dB, msgB)
        fire_scatter(msgB, idx_dB, semSB)

        @pl.when(i < NPAIR - 1)
        def _prefetch():
            drain(msgA, semSA)
            load_idx(b0 + 2, idx_sA, idx_dA)
            fire_gathers(idx_sA, idx_dA, rows_sA, rows_dA, semGA)
            drain(msgB, semSB)
            load_idx(b0 + 3, idx_sB, idx_dB)
            fire_gathers(idx_sB, idx_dB, rows_sB, rows_dB, semGB)

    drain(msgA, semSA)
    drain(msgB, semSB)
    plsc.subcore_barrier()

    @pl.when(c == 0)
    def _out0():
        pltpu.sync_copy(acc_sh.at[pl.ds(s * ROWS_PER_TILE, ROWS_PER_TILE)],
                        acc0_hbm.at[pl.ds(s * ROWS_PER_TILE, ROWS_PER_TILE)])

    @pl.when(c == 1)
    def _out1():
        pltpu.sync_copy(acc_sh.at[pl.ds(s * ROWS_PER_TILE, ROWS_PER_TILE)],
                        acc1_hbm.at[pl.ds(s * ROWS_PER_TILE, ROWS_PER_TILE)])


def _final_kernel(acc0_hbm, acc1_hbm, out_hbm, a0, a1, ob):
    c = lax.axis_index("c")
    s = lax.axis_index("s")
    wid = c * NS + s
    r0 = wid * FIN_PER_W

    pltpu.sync_copy(acc0_hbm.at[pl.ds(r0, FIN_PER_W)], a0)
    pltpu.sync_copy(acc1_hbm.at[pl.ds(r0, FIN_PER_W)], a1)

    iota = lax.iota(jnp.int32, 16)

    @pl.loop(0, FIN_STEP)
    def _step(k):
        r = k * 16 + iota
        cols = [jnp.full((16,), j, jnp.int32) for j in range(5)]
        mx = plsc.load_gather(a0, [r, cols[0]]) + plsc.load_gather(a1, [r, cols[0]])
        my = plsc.load_gather(a0, [r, cols[1]]) + plsc.load_gather(a1, [r, cols[1]])
        cx = plsc.load_gather(a0, [r, cols[2]]) + plsc.load_gather(a1, [r, cols[2]])
        cy = plsc.load_gather(a0, [r, cols[3]]) + plsc.load_gather(a1, [r, cols[3]])
        cnt = plsc.load_gather(a0, [r, cols[4]]) + plsc.load_gather(a1, [r, cols[4]])
        cnt = jnp.maximum(cnt, jnp.float32(1.0))
        ox = cx + mx / cnt
        oy = cy + my / cnt
        plsc.store_scatter(ob, [r, jnp.zeros((16,), jnp.int32)], ox)
        plsc.store_scatter(ob, [r, jnp.full((16,), 1, jnp.int32)], oy)

    pltpu.sync_copy(ob, out_hbm.at[pl.ds(r0, FIN_PER_W)])


@jax.jit
def _run(h, src2d, dst2d):
    zrows = jnp.zeros((ROWS_PER_TILE, 8), jnp.float32)
    minit = jnp.zeros((BLK, 8), jnp.float32).at[:, 4].set(1.0)

    edge_k = pl.kernel(
        _edge_kernel,
        out_type=[jax.ShapeDtypeStruct((NPAD, 8), jnp.float32),
                  jax.ShapeDtypeStruct((NPAD, 8), jnp.float32)],
        mesh=_mesh,
        scratch_types=[
            pltpu.VMEM((NSEG, SEG), jnp.int32),
            pltpu.VMEM((NSEG, SEG), jnp.int32),
            pltpu.VMEM((NSEG, SEG), jnp.int32),
            pltpu.VMEM((NSEG, SEG), jnp.int32),
            pltpu.VMEM((BLK, 8), jnp.float32),
            pltpu.VMEM((BLK, 8), jnp.float32),
            pltpu.VMEM((BLK, 8), jnp.float32),
            pltpu.VMEM((BLK, 8), jnp.float32),
            pltpu.VMEM((BLK, 8), jnp.float32),
            pltpu.VMEM((BLK, 8), jnp.float32),
            pltpu.VMEM_SHARED((NPAD, 8), jnp.float32),
            pltpu.SemaphoreType.DMA,
            pltpu.SemaphoreType.DMA,
            pltpu.SemaphoreType.DMA,
            pltpu.SemaphoreType.DMA,
        ],
        compiler_params=_cparams,
    )
    acc0, acc1 = edge_k(h, src2d, dst2d, zrows, minit)

    final_k = pl.kernel(
        _final_kernel,
        out_type=jax.ShapeDtypeStruct((NPAD, 2), jnp.float32),
        mesh=_mesh,
        scratch_types=[
            pltpu.VMEM((FIN_PER_W, 8), jnp.float32),
            pltpu.VMEM((FIN_PER_W, 8), jnp.float32),
            pltpu.VMEM((FIN_PER_W, 2), jnp.float32),
        ],
        compiler_params=_cparams,
    )
    return final_k(acc0, acc1)


def kernel(pos, vel, edge_index):
    h = jnp.concatenate([pos, vel], axis=-1)
    h = jnp.pad(h, ((0, NPAD - N_NODES), (0, 4)))
    src2d = edge_index[0].reshape(NW * NBLK, NSEG, SEG)
    dst2d = edge_index[1].reshape(NW * NBLK, NSEG, SEG)
    out = _run(h, src2d, dst2d)
    return out[:N_NODES]


# speedup vs baseline: 101.2210x; 1.9245x over previous
import dataclasses

import jax
import jax.numpy as jnp
from jax import lax
from jax.experimental import pallas as pl
from jax.experimental.pallas import tpu as pltpu
from jax.experimental.pallas import tpu_sc as plsc

N_NODES = 100000
N_EDGES = 6400000

NC = 2
NS = 16
NW = NC * NS

BLK = 1024
SEG = 128
NSEG = BLK // SEG
NSTEP = BLK // 16
NTOT = N_EDGES // BLK
NBLK_Q, NBLK_R = divmod(NTOT, NW)

NPAD = 100352
ROWS_PER_TILE = NPAD // NS
FIN_PER_W = NPAD // NW
FIN_STEP = FIN_PER_W // 16
LAST_W_ROWS = N_NODES - (NW - 1) * FIN_PER_W

_mesh = plsc.VectorSubcoreMesh(core_axis_name="c", subcore_axis_name="s")

_cparams = pltpu.CompilerParams()
for _f, _v in (("needs_layout_passes", False), ("use_tc_tiling_on_sc", False)):
    if _f in pltpu.CompilerParams.__dataclass_fields__:
        _cparams = dataclasses.replace(_cparams, **{_f: _v})


def _edge_kernel(h_hbm, ei_hbm, zrows_hbm, minit_hbm,
                 acc0_hbm, acc1_hbm,
                 idx_sA, idx_dA, idx_sB, idx_dB,
                 rows_sA, rows_dA, rows_sB, rows_dB,
                 msgA, msgB, acc_sh,
                 semGA, semGB, semSA, semSB):
    c = lax.axis_index("c")
    s = lax.axis_index("s")
    wid = c * NS + s

    pltpu.sync_copy(zrows_hbm, acc_sh.at[pl.ds(s * ROWS_PER_TILE, ROWS_PER_TILE)])
    pltpu.sync_copy(minit_hbm, msgA)
    pltpu.sync_copy(minit_hbm, msgB)
    plsc.subcore_barrier()

    iota = lax.iota(jnp.int32, 16)

    def load_idx(blk, idx_s, idx_d):
        off = pl.multiple_of(blk * BLK, BLK)
        pltpu.sync_copy(ei_hbm.at[0, pl.ds(off, BLK)], idx_s)
        pltpu.sync_copy(ei_hbm.at[1, pl.ds(off, BLK)], idx_d)

    def fire_gathers(idx_s, idx_d, rows_s, rows_d, sem):
        @pl.loop(0, NSEG)
        def _g(j):
            off = pl.multiple_of(j * SEG, SEG)
            pltpu.async_copy(h_hbm.at[idx_s.at[pl.ds(off, SEG)]],
                             rows_s.at[pl.ds(off, SEG)], sem)
            pltpu.async_copy(h_hbm.at[idx_d.at[pl.ds(off, SEG)]],
                             rows_d.at[pl.ds(off, SEG)], sem)

    def drain(buf, sem):
        pltpu.make_async_copy(h_hbm.at[pl.ds(0, BLK)], buf, sem).wait()

    def compute(rows_s, rows_d, msg):
        @pl.loop(0, NSTEP)
        def _compute(k):
            r = k * 16 + iota
            s0 = plsc.load_gather(rows_s, [r, jnp.zeros((16,), jnp.int32)])
            s1 = plsc.load_gather(rows_s, [r, jnp.full((16,), 1, jnp.int32)])
            s2 = plsc.load_gather(rows_s, [r, jnp.full((16,), 2, jnp.int32)])
            s3 = plsc.load_gather(rows_s, [r, jnp.full((16,), 3, jnp.int32)])
            d0 = plsc.load_gather(rows_d, [r, jnp.zeros((16,), jnp.int32)])
            d1 = plsc.load_gather(rows_d, [r, jnp.full((16,), 1, jnp.int32)])
            d2 = plsc.load_gather(rows_d, [r, jnp.full((16,), 2, jnp.int32)])
            d3 = plsc.load_gather(rows_d, [r, jnp.full((16,), 3, jnp.int32)])
            px = s0 - d0
            py = s1 - d1
            vx = s2 - d2
            vy = s3 - d3
            nsq = px * px + py * py
            inv = jnp.float32(1.0) / nsq
            zero = nsq == jnp.float32(0.0)
            cx = jnp.where(zero, jnp.float32(0.0), px * jnp.float32(-5.0) * inv)
            cy = jnp.where(zero, jnp.float32(0.0), py * jnp.float32(-5.0) * inv)
            mx = px * jnp.float32(0.05) + vx
            my = py * jnp.float32(0.05) + vy
            plsc.store_scatter(msg, [r, jnp.zeros((16,), jnp.int32)], mx)
            plsc.store_scatter(msg, [r, jnp.full((16,), 1, jnp.int32)], my)
            plsc.store_scatter(msg, [r, jnp.full((16,), 2, jnp.int32)], cx)
            plsc.store_scatter(msg, [r, jnp.full((16,), 3, jnp.int32)], cy)

    def fire_scatter(msg, idx_d, sem):
        @pl.loop(0, NSEG)
        def _s(j):
            off = pl.multiple_of(j * SEG, SEG)
            pltpu.async_copy(msg.at[pl.ds(off, SEG)],
                             acc_sh.at[idx_d.at[pl.ds(off, SEG)]],
                             sem, add=True)

    nblk = jnp.where(wid < NBLK_R, NBLK_Q + 1, NBLK_Q)
    base = wid * NBLK_Q + jnp.minimum(wid, NBLK_R)
    npair = nblk // 2

    load_idx(base + 0, idx_sA, idx_dA)
    fire_gathers(idx_sA, idx_dA, rows_sA, rows_dA, semGA)
    load_idx(base + 1, idx_sB, idx_dB)
    fire_gathers(idx_sB, idx_dB, rows_sB, rows_dB, semGB)

    @pl.loop(0, npair)
    def _pair(i):
        b0 = base + 2 * i
        drain(rows_sA, semGA)
        drain(rows_dA, semGA)
        compute(rows_sA, rows_dA, msgA)
        fire_scatter(msgA, idx_dA, semSA)
        drain(rows_sB, semGB)
        drain(rows_dB, semGB)
        compute(rows_sB, rows_dB, msgB)
        fire_scatter(msgB, idx_dB, semSB)

        @pl.when(2 * i + 2 < nblk)
        def _prefA():
            drain(msgA, semSA)
            load_idx(b0 + 2, idx_sA, idx_dA)
            fire_gathers(idx_sA, idx_dA, rows_sA, rows_dA, semGA)

        @pl.when(2 * i + 3 < nblk)
        def _prefB():
            drain(msgB, semSB)
            load_idx(b0 + 3, idx_sB, idx_dB)
            fire_gathers(idx_sB, idx_dB, rows_sB, rows_dB, semGB)

    @pl.when(nblk % 2 == 1)
    def _tail():
        drain(rows_sA, semGA)
        drain(rows_dA, semGA)
        compute(rows_sA, rows_dA, msgA)
        fire_scatter(msgA, idx_dA, semSA)

    drain(msgA, semSA)
    drain(msgB, semSB)
    plsc.subcore_barrier()

    @pl.when(c == 0)
    def _out0():
        pltpu.sync_copy(acc_sh.at[pl.ds(s * ROWS_PER_TILE, ROWS_PER_TILE)],
                        acc0_hbm.at[pl.ds(s * ROWS_PER_TILE, ROWS_PER_TILE)])

    @pl.when(c == 1)
    def _out1():
        pltpu.sync_copy(acc_sh.at[pl.ds(s * ROWS_PER_TILE, ROWS_PER_TILE)],
                        acc1_hbm.at[pl.ds(s * ROWS_PER_TILE, ROWS_PER_TILE)])


def _final_kernel(acc0_hbm, acc1_hbm, out_hbm, a0, a1, ob):
    c = lax.axis_index("c")
    s = lax.axis_index("s")
    wid = c * NS + s
    r0 = wid * FIN_PER_W

    pltpu.sync_copy(acc0_hbm.at[pl.ds(r0, FIN_PER_W)], a0)
    pltpu.sync_copy(acc1_hbm.at[pl.ds(r0, FIN_PER_W)], a1)

    iota = lax.iota(jnp.int32, 16)

    @pl.loop(0, FIN_STEP)
    def _step(k):
        r = k * 16 + iota
        cols = [jnp.full((16,), j, jnp.int32) for j in range(5)]
        mx = plsc.load_gather(a0, [r, cols[0]]) + plsc.load_gather(a1, [r, cols[0]])
        my = plsc.load_gather(a0, [r, cols[1]]) + plsc.load_gather(a1, [r, cols[1]])
        cx = plsc.load_gather(a0, [r, cols[2]]) + plsc.load_gather(a1, [r, cols[2]])
        cy = plsc.load_gather(a0, [r, cols[3]]) + plsc.load_gather(a1, [r, cols[3]])
        cnt = plsc.load_gather(a0, [r, cols[4]]) + plsc.load_gather(a1, [r, cols[4]])
        cnt = jnp.maximum(cnt, jnp.float32(1.0))
        ox = cx + mx / cnt
        oy = cy + my / cnt
        plsc.store_scatter(ob, [r, jnp.zeros((16,), jnp.int32)], ox)
        plsc.store_scatter(ob, [r, jnp.full((16,), 1, jnp.int32)], oy)

    @pl.when(wid < NW - 1)
    def _full():
        pltpu.sync_copy(ob, out_hbm.at[pl.ds(r0, FIN_PER_W)])

    @pl.when(wid == NW - 1)
    def _last():
        pltpu.sync_copy(ob.at[pl.ds(0, LAST_W_ROWS)],
                        out_hbm.at[pl.ds(r0, LAST_W_ROWS)])


@jax.jit
def _run(h, edge_index):
    zrows = jnp.zeros((ROWS_PER_TILE, 8), jnp.float32)
    minit = jnp.zeros((BLK, 8), jnp.float32).at[:, 4].set(1.0)

    edge_k = pl.kernel(
        _edge_kernel,
        out_type=[jax.ShapeDtypeStruct((NPAD, 8), jnp.float32),
                  jax.ShapeDtypeStruct((NPAD, 8), jnp.float32)],
        mesh=_mesh,
        scratch_types=[
            pltpu.VMEM((BLK,), jnp.int32),
            pltpu.VMEM((BLK,), jnp.int32),
            pltpu.VMEM((BLK,), jnp.int32),
            pltpu.VMEM((BLK,), jnp.int32),
            pltpu.VMEM((BLK, 8), jnp.float32),
            pltpu.VMEM((BLK, 8), jnp.float32),
            pltpu.VMEM((BLK, 8), jnp.float32),
            pltpu.VMEM((BLK, 8), jnp.float32),
            pltpu.VMEM((BLK, 8), jnp.float32),
            pltpu.VMEM((BLK, 8), jnp.float32),
            pltpu.VMEM_SHARED((NPAD, 8), jnp.float32),
            pltpu.SemaphoreType.DMA,
            pltpu.SemaphoreType.DMA,
            pltpu.SemaphoreType.DMA,
            pltpu.SemaphoreType.DMA,
        ],
        compiler_params=_cparams,
    )
    acc0, acc1 = edge_k(h, edge_index, zrows, minit)

    final_k = pl.kernel(
        _final_kernel,
        out_type=jax.ShapeDtypeStruct((N_NODES, 2), jnp.float32),
        mesh=_mesh,
        scratch_types=[
            pltpu.VMEM((FIN_PER_W, 8), jnp.float32),
            pltpu.VMEM((FIN_PER_W, 8), jnp.float32),
            pltpu.VMEM((FIN_PER_W, 2), jnp.float32),
        ],
        compiler_params=_cparams,
    )
    return final_k(acc0, acc1)


def kernel(pos, vel, edge_index):
    h = jnp.concatenate([pos, vel], axis=-1)
    h = jnp.pad(h, ((0, NPAD - N_NODES), (0, 4)))
    return _run(h, edge_index)


# speedup vs baseline: 117.9411x; 1.1652x over previous
import dataclasses

import jax
import jax.numpy as jnp
from jax import lax
from jax.experimental import pallas as pl
from jax.experimental.pallas import tpu as pltpu
from jax.experimental.pallas import tpu_sc as plsc

N_NODES = 100000
N_EDGES = 6400000

NC = 2
NS = 16
NW = NC * NS

BLK = 1024
SEG = 128
NSEG = BLK // SEG
NSTEP = BLK // 16
NTOT = N_EDGES // BLK
NBLK_Q, NBLK_R = divmod(NTOT, NW)

NPAD = 100352
ROWS_PER_TILE = NPAD // NS
FIN_PER_W = NPAD // NW
FIN_STEP = FIN_PER_W // 16
LAST_W_ROWS = N_NODES - (NW - 1) * FIN_PER_W

_mesh = plsc.VectorSubcoreMesh(core_axis_name="c", subcore_axis_name="s")

_cparams = pltpu.CompilerParams()
for _f, _v in (("needs_layout_passes", False), ("use_tc_tiling_on_sc", False)):
    if _f in pltpu.CompilerParams.__dataclass_fields__:
        _cparams = dataclasses.replace(_cparams, **{_f: _v})


NSLOT = 3
NTRI = NBLK_Q // NSLOT


def _edge_kernel(h_hbm, ei_hbm, zrows_hbm, minit_hbm,
                 acc0_hbm, acc1_hbm,
                 idx_s0, idx_d0, idx_s1, idx_d1, idx_s2, idx_d2,
                 rows_s0, rows_d0, rows_s1, rows_d1, rows_s2, rows_d2,
                 msg0, msg1, msg2, acc_sh,
                 semG0, semG1, semG2, semS0, semS1, semS2):
    c = lax.axis_index("c")
    s = lax.axis_index("s")
    wid = c * NS + s

    slots = [
        (idx_s0, idx_d0, rows_s0, rows_d0, msg0, semG0, semS0),
        (idx_s1, idx_d1, rows_s1, rows_d1, msg1, semG1, semS1),
        (idx_s2, idx_d2, rows_s2, rows_d2, msg2, semG2, semS2),
    ]

    pltpu.sync_copy(zrows_hbm, acc_sh.at[pl.ds(s * ROWS_PER_TILE, ROWS_PER_TILE)])
    for sl in slots:
        pltpu.sync_copy(minit_hbm, sl[4])
    plsc.subcore_barrier()

    iota = lax.iota(jnp.int32, 16)

    def load_idx(blk, idx_s, idx_d):
        off = pl.multiple_of(blk * BLK, BLK)
        pltpu.sync_copy(ei_hbm.at[0, pl.ds(off, BLK)], idx_s)
        pltpu.sync_copy(ei_hbm.at[1, pl.ds(off, BLK)], idx_d)

    def fire_gathers(idx_s, idx_d, rows_s, rows_d, sem):
        @pl.loop(0, NSEG)
        def _g(j):
            off = pl.multiple_of(j * SEG, SEG)
            pltpu.async_copy(h_hbm.at[idx_s.at[pl.ds(off, SEG)]],
                             rows_s.at[pl.ds(off, SEG)], sem)
            pltpu.async_copy(h_hbm.at[idx_d.at[pl.ds(off, SEG)]],
                             rows_d.at[pl.ds(off, SEG)], sem)

    def drain(buf, sem):
        pltpu.make_async_copy(h_hbm.at[pl.ds(0, BLK)], buf, sem).wait()

    def compute(rows_s, rows_d, msg):
        @pl.loop(0, NSTEP)
        def _compute(k):
            r = k * 16 + iota
            s0 = plsc.load_gather(rows_s, [r, jnp.zeros((16,), jnp.int32)])
            s1 = plsc.load_gather(rows_s, [r, jnp.full((16,), 1, jnp.int32)])
            s2 = plsc.load_gather(rows_s, [r, jnp.full((16,), 2, jnp.int32)])
            s3 = plsc.load_gather(rows_s, [r, jnp.full((16,), 3, jnp.int32)])
            d0 = plsc.load_gather(rows_d, [r, jnp.zeros((16,), jnp.int32)])
            d1 = plsc.load_gather(rows_d, [r, jnp.full((16,), 1, jnp.int32)])
            d2 = plsc.load_gather(rows_d, [r, jnp.full((16,), 2, jnp.int32)])
            d3 = plsc.load_gather(rows_d, [r, jnp.full((16,), 3, jnp.int32)])
            px = s0 - d0
            py = s1 - d1
            vx = s2 - d2
            vy = s3 - d3
            nsq = px * px + py * py
            inv = jnp.float32(1.0) / nsq
            zero = nsq == jnp.float32(0.0)
            cx = jnp.where(zero, jnp.float32(0.0), px * jnp.float32(-5.0) * inv)
            cy = jnp.where(zero, jnp.float32(0.0), py * jnp.float32(-5.0) * inv)
            mx = px * jnp.float32(0.05) + vx
            my = py * jnp.float32(0.05) + vy
            plsc.store_scatter(msg, [r, jnp.zeros((16,), jnp.int32)], mx)
            plsc.store_scatter(msg, [r, jnp.full((16,), 1, jnp.int32)], my)
            plsc.store_scatter(msg, [r, jnp.full((16,), 2, jnp.int32)], cx)
            plsc.store_scatter(msg, [r, jnp.full((16,), 3, jnp.int32)], cy)

    def fire_scatter(msg, idx_d, sem):
        @pl.loop(0, NSEG)
        def _s(j):
            off = pl.multiple_of(j * SEG, SEG)
            pltpu.async_copy(msg.at[pl.ds(off, SEG)],
                             acc_sh.at[idx_d.at[pl.ds(off, SEG)]],
                             sem, add=True)

    nblk = jnp.where(wid < NBLK_R, NBLK_Q + 1, NBLK_Q)
    base = wid * NBLK_Q + jnp.minimum(wid, NBLK_R)

    def process(sl):
        idx_s, idx_d, rows_s, rows_d, msg, semG, semS = sl
        drain(rows_s, semG)
        drain(rows_d, semG)
        compute(rows_s, rows_d, msg)
        fire_scatter(msg, idx_d, semS)

    def prefetch(sl, blk, cond):
        idx_s, idx_d, rows_s, rows_d, msg, semG, semS = sl

        @pl.when(cond)
        def _p():
            drain(msg, semS)
            load_idx(blk, idx_s, idx_d)
            fire_gathers(idx_s, idx_d, rows_s, rows_d, semG)

    for k, sl in enumerate(slots):
        load_idx(base + k, sl[0], sl[1])
        fire_gathers(sl[0], sl[1], sl[2], sl[3], sl[5])

    @pl.loop(0, NTRI)
    def _tri(i):
        b0 = base + NSLOT * i
        process(slots[0])
        process(slots[1])
        prefetch(slots[0], b0 + NSLOT, NSLOT * i + NSLOT < nblk)
        process(slots[2])
        prefetch(slots[1], b0 + NSLOT + 1, NSLOT * i + NSLOT + 1 < nblk)
        prefetch(slots[2], b0 + NSLOT + 2, NSLOT * i + NSLOT + 2 < nblk)

    @pl.when(nblk % NSLOT == 1)
    def _tail():
        process(slots[0])

    for sl in slots:
        drain(sl[4], sl[6])
    plsc.subcore_barrier()

    @pl.when(c == 0)
    def _out0():
        pltpu.sync_copy(acc_sh.at[pl.ds(s * ROWS_PER_TILE, ROWS_PER_TILE)],
                        acc0_hbm.at[pl.ds(s * ROWS_PER_TILE, ROWS_PER_TILE)])

    @pl.when(c == 1)
    def _out1():
        pltpu.sync_copy(acc_sh.at[pl.ds(s * ROWS_PER_TILE, ROWS_PER_TILE)],
                        acc1_hbm.at[pl.ds(s * ROWS_PER_TILE, ROWS_PER_TILE)])


def _final_kernel(acc0_hbm, acc1_hbm, out_hbm, a0, a1, ob):
    c = lax.axis_index("c")
    s = lax.axis_index("s")
    wid = c * NS + s
    r0 = wid * FIN_PER_W

    pltpu.sync_copy(acc0_hbm.at[pl.ds(r0, FIN_PER_W)], a0)
    pltpu.sync_copy(acc1_hbm.at[pl.ds(r0, FIN_PER_W)], a1)

    iota = lax.iota(jnp.int32, 16)

    @pl.loop(0, FIN_STEP)
    def _step(k):
        r = k * 16 + iota
        cols = [jnp.full((16,), j, jnp.int32) for j in range(5)]
        mx = plsc.load_gather(a0, [r, cols[0]]) + plsc.load_gather(a1, [r, cols[0]])
        my = plsc.load_gather(a0, [r, cols[1]]) + plsc.load_gather(a1, [r, cols[1]])
        cx = plsc.load_gather(a0, [r, cols[2]]) + plsc.load_gather(a1, [r, cols[2]])
        cy = plsc.load_gather(a0, [r, cols[3]]) + plsc.load_gather(a1, [r, cols[3]])
        cnt = plsc.load_gather(a0, [r, cols[4]]) + plsc.load_gather(a1, [r, cols[4]])
        cnt = jnp.maximum(cnt, jnp.float32(1.0))
        ox = cx + mx / cnt
        oy = cy + my / cnt
        plsc.store_scatter(ob, [r, jnp.zeros((16,), jnp.int32)], ox)
        plsc.store_scatter(ob, [r, jnp.full((16,), 1, jnp.int32)], oy)

    @pl.when(wid < NW - 1)
    def _full():
        pltpu.sync_copy(ob, out_hbm.at[pl.ds(r0, FIN_PER_W)])

    @pl.when(wid == NW - 1)
    def _last():
        pltpu.sync_copy(ob.at[pl.ds(0, LAST_W_ROWS)],
                        out_hbm.at[pl.ds(r0, LAST_W_ROWS)])


@jax.jit
def _run(h, edge_index):
    zrows = jnp.zeros((ROWS_PER_TILE, 8), jnp.float32)
    minit = jnp.zeros((BLK, 8), jnp.float32).at[:, 4].set(1.0)

    edge_k = pl.kernel(
        _edge_kernel,
        out_type=[jax.ShapeDtypeStruct((NPAD, 8), jnp.float32),
                  jax.ShapeDtypeStruct((NPAD, 8), jnp.float32)],
        mesh=_mesh,
        scratch_types=(
            [pltpu.VMEM((BLK,), jnp.int32)] * 6
            + [pltpu.VMEM((BLK, 8), jnp.float32)] * 9
            + [pltpu.VMEM_SHARED((NPAD, 8), jnp.float32)]
            + [pltpu.SemaphoreType.DMA] * 6
        ),
        compiler_params=_cparams,
    )
    acc0, acc1 = edge_k(h, edge_index, zrows, minit)

    final_k = pl.kernel(
        _final_kernel,
        out_type=jax.ShapeDtypeStruct((N_NODES, 2), jnp.float32),
        mesh=_mesh,
        scratch_types=[
            pltpu.VMEM((FIN_PER_W, 8), jnp.float32),
            pltpu.VMEM((FIN_PER_W, 8), jnp.float32),
            pltpu.VMEM((FIN_PER_W, 2), jnp.float32),
        ],
        compiler_params=_cparams,
    )
    return final_k(acc0, acc1)


def kernel(pos, vel, edge_index):
    h = jnp.concatenate([pos, vel], axis=-1)
    h = jnp.pad(h, ((0, NPAD - N_NODES), (0, 4)))
    return _run(h, edge_index)
